# baseline (speedup 1.0000x reference)
import jax
import jax.numpy as jnp
from jax import lax
from jax.experimental import pallas as pl
from jax.experimental.pallas import tpu as pltpu

_E = 8
_H = 1024
_I = 512
_ZC = 0.01
_T = 2048
_TT = 2048
_IC = 512


def _body(x_ref, gw_ref, wg_ref, wu_ref, wd_ref, out_ref, logits_ref, scores_s, logits_s):
    t = pl.program_id(0)
    e = pl.program_id(1)

    @pl.when(e == 0)
    def _router():
        logits = lax.dot_general(
            x_ref[...], gw_ref[...], (((1,), (1,)), ((), ())),
            preferred_element_type=jnp.float32,
        )
        logits_s[...] = logits
        logits_ref[...] = logits_s[pl.ds(t * _TT, _TT), :]
        m = jnp.mean(logits, axis=-1, keepdims=True)
        c = logits - m
        z_loss = _ZC * jnp.sum(c * c) / (_T * _E)
        l2 = logits - z_loss * logits
        rowmax = jnp.max(l2, axis=-1, keepdims=True)
        p = jnp.exp(l2 - rowmax)
        p = p / jnp.sum(p, axis=-1, keepdims=True)
        scores_s[...] = p / jnp.clip(jnp.sum(p, axis=-1, keepdims=True), 1e-8, None)

    x = x_ref[pl.ds(t * _TT, _TT), :]
    lane = lax.broadcasted_iota(jnp.int32, (1, _E), 1)
    s_col = jnp.sum(
        jnp.where(lane == e, scores_s[pl.ds(t * _TT, _TT), :], 0.0),
        axis=-1,
        keepdims=True,
    )

    o = None
    for i0 in range(0, _I, _IC):
        g = lax.dot_general(
            x, wg_ref[0, pl.ds(i0, _IC), :], (((1,), (1,)), ((), ())),
            preferred_element_type=jnp.float32,
        )
        u = lax.dot_general(
            x, wu_ref[0, pl.ds(i0, _IC), :], (((1,), (1,)), ((), ())),
            preferred_element_type=jnp.float32,
        )
        h = (g * jax.nn.sigmoid(g) * u) * s_col
        oc = lax.dot_general(
            h, wd_ref[0, :, pl.ds(i0, _IC)], (((1,), (1,)), ((), ())),
            preferred_element_type=jnp.float32,
        )
        o = oc if o is None else o + oc

    @pl.when(e == 0)
    def _():
        out_ref[...] = o

    @pl.when(e != 0)
    def _():
        out_ref[...] += o


@jax.jit
def kernel(hidden_states, gate_w, w_gate, w_up, w_down):
    B, S, H = hidden_states.shape
    T = B * S
    x = hidden_states.reshape(T, H)

    n_t = T // _TT
    final, router_logits = pl.pallas_call(
        _body,
        grid=(n_t, _E),
        in_specs=[
            pl.BlockSpec((T, _H), lambda t, e: (0, 0)),
            pl.BlockSpec((_E, _H), lambda t, e: (0, 0)),
            pl.BlockSpec((1, _I, _H), lambda t, e: (e, 0, 0)),
            pl.BlockSpec((1, _I, _H), lambda t, e: (e, 0, 0)),
            pl.BlockSpec((1, _H, _I), lambda t, e: (e, 0, 0)),
        ],
        out_specs=(
            pl.BlockSpec((_TT, _H), lambda t, e: (t, 0)),
            pl.BlockSpec((_TT, _E), lambda t, e: (t, 0)),
        ),
        out_shape=(
            jax.ShapeDtypeStruct((T, _H), jnp.float32),
            jax.ShapeDtypeStruct((T, _E), jnp.float32),
        ),
        scratch_shapes=[
            pltpu.VMEM((T, _E), jnp.float32),
            pltpu.VMEM((T, _E), jnp.float32),
        ],
        compiler_params=pltpu.CompilerParams(
            dimension_semantics=("parallel", "arbitrary"),
        ),
    )(x, gate_w, w_gate, w_up, w_down)

    return final.reshape(B, S, H), router_logits


# speedup vs baseline: 1.0056x; 1.0056x over previous
import jax
import jax.numpy as jnp
from jax import lax
from jax.experimental import pallas as pl
from jax.experimental.pallas import tpu as pltpu

_E = 8
_H = 1024
_I = 512
_ZC = 0.01
_T = 2048
_TT = 2048
_IC = 256


def _body(x_ref, gw_ref, wg_ref, wu_ref, wd_ref, out_ref, logits_ref, scores_s, logits_s, xb_s):
    t = pl.program_id(0)
    e = pl.program_id(1)

    @pl.when(e == 0)
    def _router():
        xb_s[...] = x_ref[...].astype(jnp.bfloat16)
        logits = lax.dot_general(
            x_ref[...], gw_ref[...], (((1,), (1,)), ((), ())),
            preferred_element_type=jnp.float32,
        )
        logits_s[...] = logits
        logits_ref[...] = logits_s[pl.ds(t * _TT, _TT), :]
        m = jnp.mean(logits, axis=-1, keepdims=True)
        c = logits - m
        z_loss = _ZC * jnp.sum(c * c) / (_T * _E)
        l2 = logits - z_loss * logits
        rowmax = jnp.max(l2, axis=-1, keepdims=True)
        p = jnp.exp(l2 - rowmax)
        p = p / jnp.sum(p, axis=-1, keepdims=True)
        scores_s[...] = p / jnp.clip(jnp.sum(p, axis=-1, keepdims=True), 1e-8, None)

    x = xb_s[pl.ds(t * _TT, _TT), :]
    lane = lax.broadcasted_iota(jnp.int32, (1, _E), 1)
    s_col = jnp.sum(
        jnp.where(lane == e, scores_s[pl.ds(t * _TT, _TT), :], 0.0),
        axis=-1,
        keepdims=True,
    )

    o = None
    for i0 in range(0, _I, _IC):
        g = lax.dot_general(
            x, wg_ref[0, pl.ds(i0, _IC), :].astype(jnp.bfloat16),
            (((1,), (1,)), ((), ())), preferred_element_type=jnp.float32,
        )
        u = lax.dot_general(
            x, wu_ref[0, pl.ds(i0, _IC), :].astype(jnp.bfloat16),
            (((1,), (1,)), ((), ())), preferred_element_type=jnp.float32,
        )
        h = (g * jax.nn.sigmoid(g) * u) * s_col
        oc = lax.dot_general(
            h.astype(jnp.bfloat16), wd_ref[0, :, pl.ds(i0, _IC)].astype(jnp.bfloat16),
            (((1,), (1,)), ((), ())), preferred_element_type=jnp.float32,
        )
        o = oc if o is None else o + oc

    @pl.when(e == 0)
    def _():
        out_ref[...] = o

    @pl.when(e != 0)
    def _():
        out_ref[...] += o


@jax.jit
def kernel(hidden_states, gate_w, w_gate, w_up, w_down):
    B, S, H = hidden_states.shape
    T = B * S
    x = hidden_states.reshape(T, H)

    n_t = T // _TT
    final, router_logits = pl.pallas_call(
        _body,
        grid=(n_t, _E),
        in_specs=[
            pl.BlockSpec((T, _H), lambda t, e: (0, 0)),
            pl.BlockSpec((_E, _H), lambda t, e: (0, 0)),
            pl.BlockSpec((1, _I, _H), lambda t, e: (e, 0, 0)),
            pl.BlockSpec((1, _I, _H), lambda t, e: (e, 0, 0)),
            pl.BlockSpec((1, _H, _I), lambda t, e: (e, 0, 0)),
        ],
        out_specs=(
            pl.BlockSpec((_TT, _H), lambda t, e: (t, 0)),
            pl.BlockSpec((_TT, _E), lambda t, e: (t, 0)),
        ),
        out_shape=(
            jax.ShapeDtypeStruct((T, _H), jnp.float32),
            jax.ShapeDtypeStruct((T, _E), jnp.float32),
        ),
        scratch_shapes=[
            pltpu.VMEM((T, _E), jnp.float32),
            pltpu.VMEM((T, _E), jnp.float32),
            pltpu.VMEM((T, _H), jnp.bfloat16),
        ],
        compiler_params=pltpu.CompilerParams(
            dimension_semantics=("parallel", "arbitrary"),
        ),
    )(x, gate_w, w_gate, w_up, w_down)

    return final.reshape(B, S, H), router_logits


# speedup vs baseline: 1.0341x; 1.0284x over previous
import jax
import jax.numpy as jnp
from jax import lax
from jax.experimental import pallas as pl
from jax.experimental.pallas import tpu as pltpu

_E = 8
_H = 1024
_I = 512
_ZC = 0.01
_T = 2048
_IC = 256


def _body(x_ref, gw_ref, wg_ref, wu_ref, wd_ref, out_ref, logits_ref, scores_s):
    e = pl.program_id(0)

    @pl.when(e == 0)
    def _router():
        logits = lax.dot_general(
            x_ref[...], gw_ref[...], (((1,), (1,)), ((), ())),
            preferred_element_type=jnp.float32,
        )
        logits_ref[...] = logits
        m = jnp.mean(logits, axis=-1, keepdims=True)
        c = logits - m
        z_loss = _ZC * jnp.sum(c * c) / (_T * _E)
        l2 = logits - z_loss * logits
        rowmax = jnp.max(l2, axis=-1, keepdims=True)
        p = jnp.exp(l2 - rowmax)
        p = p / jnp.sum(p, axis=-1, keepdims=True)
        scores_s[...] = p / jnp.clip(jnp.sum(p, axis=-1, keepdims=True), 1e-8, None)

    x = x_ref[...]
    lane = lax.broadcasted_iota(jnp.int32, (1, _E), 1)
    s_col = jnp.sum(
        jnp.where(lane == e, scores_s[...], 0.0), axis=-1, keepdims=True
    )

    o = None
    for i0 in range(0, _I, _IC):
        g = lax.dot_general(
            x, wg_ref[0, pl.ds(i0, _IC), :], (((1,), (1,)), ((), ())),
            preferred_element_type=jnp.float32,
        )
        u = lax.dot_general(
            x, wu_ref[0, pl.ds(i0, _IC), :], (((1,), (1,)), ((), ())),
            preferred_element_type=jnp.float32,
        )
        h = (g * jax.nn.sigmoid(g) * u) * s_col
        oc = lax.dot_general(
            h, wd_ref[0, :, pl.ds(i0, _IC)], (((1,), (1,)), ((), ())),
            preferred_element_type=jnp.float32,
        )
        o = oc if o is None else o + oc

    @pl.when(e == 0)
    def _():
        out_ref[...] = o

    @pl.when(e != 0)
    def _():
        out_ref[...] += o


@jax.jit
def kernel(hidden_states, gate_w, w_gate, w_up, w_down):
    B, S, H = hidden_states.shape
    T = B * S
    x = hidden_states.reshape(T, H)

    final, router_logits = pl.pallas_call(
        _body,
        grid=(_E,),
        in_specs=[
            pl.BlockSpec((T, _H), lambda e: (0, 0)),
            pl.BlockSpec((_E, _H), lambda e: (0, 0)),
            pl.BlockSpec((1, _I, _H), lambda e: (e, 0, 0)),
            pl.BlockSpec((1, _I, _H), lambda e: (e, 0, 0)),
            pl.BlockSpec((1, _H, _I), lambda e: (e, 0, 0)),
        ],
        out_specs=(
            pl.BlockSpec((T, _H), lambda e: (0, 0)),
            pl.BlockSpec((T, _E), lambda e: (0, 0)),
        ),
        out_shape=(
            jax.ShapeDtypeStruct((T, _H), jnp.float32),
            jax.ShapeDtypeStruct((T, _E), jnp.float32),
        ),
        scratch_shapes=[pltpu.VMEM((T, _E), jnp.float32)],
        compiler_params=pltpu.CompilerParams(
            dimension_semantics=("arbitrary",),
        ),
    )(x, gate_w, w_gate, w_up, w_down)

    return final.reshape(B, S, H), router_logits
